# Initial kernel scaffold; baseline (speedup 1.0000x reference)
import jax
import jax.numpy as jnp
from jax.experimental import pallas as pl


def kernel(pred_boxes, targets):
    raise NotImplementedError("write your pallas kernel here")


# speedup vs baseline: 5.4814x; 5.4814x over previous
import jax
import jax.numpy as jnp
from jax.experimental import pallas as pl
from jax.experimental.pallas import tpu as pltpu

_INF = 100000000.0
_TOPK = 9


def _atss_kernel(ax0, ay0, ax1, ay1, gx0, gy0, gx1, gy1, out):
    a0, b0, a1, b1 = ax0[...], ay0[...], ax1[...], ay1[...]
    g0, h0, g1, h1 = gx0[...], gy0[...], gx1[...], gy1[...]
    N = a0.shape[1]
    G = g0.shape[0]

    acx = (a0 + a1) * 0.5
    acy = (b0 + b1) * 0.5
    gcx = (g0 + g1) * 0.5
    gcy = (h0 + h1) * 0.5

    area_a = (a1 - a0) * (b1 - b0)
    area_g = (g1 - g0) * (h1 - h0)
    iw = jnp.maximum(jnp.minimum(a1, g1) - jnp.maximum(a0, g0), 0.0)
    ih = jnp.maximum(jnp.minimum(b1, h1) - jnp.maximum(b0, h0), 0.0)
    inter = iw * ih
    iou = inter / (area_a + area_g - inter)

    dx = acx - gcx
    dy = acy - gcy
    dist = jnp.sqrt(dx * dx + dy * dy)

    lane = jax.lax.broadcasted_iota(jnp.int32, (G, N), 1)
    cur = dist
    mask = jnp.zeros((G, N), dtype=jnp.bool_)
    sum_iou = jnp.zeros((G, 1), dtype=jnp.float32)
    for _ in range(_TOPK):
        m = jnp.min(cur, axis=1, keepdims=True)
        idx = jnp.min(jnp.where(cur == m, lane, N), axis=1, keepdims=True)
        onehot = lane == idx
        sum_iou = sum_iou + jnp.sum(jnp.where(onehot, iou, 0.0), axis=1,
                                    keepdims=True)
        mask = mask | onehot
        cur = jnp.where(onehot, _INF, cur)

    mean = sum_iou * (1.0 / _TOPK)
    var = jnp.sum(jnp.where(mask, (iou - mean) ** 2, 0.0), axis=1,
                  keepdims=True) * (1.0 / (_TOPK - 1))
    thresh = mean + jnp.sqrt(var)

    in_box = (
        jnp.minimum(jnp.minimum(acx - g0, acy - h0),
                    jnp.minimum(g1 - acx, h1 - acy)) > 0.01)
    pos = mask & (iou >= thresh) & in_box
    masked = jnp.where(pos, iou, -_INF)

    best_v = jnp.max(masked, axis=0, keepdims=True)
    grow = jax.lax.broadcasted_iota(jnp.int32, (G, N), 0)
    best_g = jnp.min(jnp.where(masked == best_v, grow, G), axis=0,
                     keepdims=True)
    sel = grow == best_g
    zero = jnp.zeros((G, N), dtype=jnp.float32)
    out[0:1, :] = jnp.sum(jnp.where(sel, g0 + zero, 0.0), axis=0, keepdims=True)
    out[1:2, :] = jnp.sum(jnp.where(sel, h0 + zero, 0.0), axis=0, keepdims=True)
    out[2:3, :] = jnp.sum(jnp.where(sel, g1 + zero, 0.0), axis=0, keepdims=True)
    out[3:4, :] = jnp.sum(jnp.where(sel, h1 + zero, 0.0), axis=0, keepdims=True)
    out[4:5, :] = best_v


def kernel(pred_boxes, targets):
    anchors = pred_boxes[0]
    N = anchors.shape[0]
    bboxes = targets[:, 1:-1]
    G = bboxes.shape[0]

    ax0 = anchors[:, 0].reshape(1, N)
    ay0 = anchors[:, 1].reshape(1, N)
    ax1 = anchors[:, 2].reshape(1, N)
    ay1 = anchors[:, 3].reshape(1, N)
    gx0 = bboxes[:, 0].reshape(G, 1)
    gy0 = bboxes[:, 1].reshape(G, 1)
    gx1 = bboxes[:, 2].reshape(G, 1)
    gy1 = bboxes[:, 3].reshape(G, 1)

    out = pl.pallas_call(
        _atss_kernel,
        out_shape=jax.ShapeDtypeStruct((5, N), jnp.float32),
    )(ax0, ay0, ax1, ay1, gx0, gy0, gx1, gy1)
    return out.T
